# baseline (speedup 1.0000x reference)
import jax
import jax.numpy as jnp
from jax import lax
from jax.experimental import pallas as pl
from jax.experimental.pallas import tpu as pltpu

_CH = 200
_NBUF = 4


def _gcn_kernel(x_ref, w_ref, b_ref, adj_ref, support_ref, hidden_ref,
                out_ref, bufs, sems):
    i = pl.program_id(0)
    nsteps = pl.num_programs(0)

    def start(chunk, slot):
        pltpu.make_async_copy(
            adj_ref.at[pl.ds(chunk * _CH, _CH), :],
            bufs.at[slot],
            sems.at[slot],
        ).start()

    @pl.when(i == 0)
    def _():
        for s in range(_NBUF):
            start(s, s)
        support_ref[...] = jnp.dot(
            x_ref[...], w_ref[...], preferred_element_type=jnp.float32
        )

    slot = lax.rem(i, _NBUF)
    pltpu.make_async_copy(
        adj_ref.at[pl.ds(i * _CH, _CH), :],
        bufs.at[slot],
        sems.at[slot],
    ).wait()

    acc = jnp.dot(
        bufs[slot], support_ref[...], preferred_element_type=jnp.float32
    )
    acc = acc + b_ref[...]
    out_ref[...] = acc
    hidden_ref[...] = jnp.maximum(acc, 0.0)

    @pl.when(i + _NBUF < nsteps)
    def _():
        start(i + _NBUF, slot)


def kernel(x, adj, grad_adj, W, b):
    N, din = x.shape
    dout = W.shape[1]
    grid = (N // _CH,)

    b2 = b.reshape(1, dout)

    support, hidden, output = pl.pallas_call(
        _gcn_kernel,
        grid=grid,
        in_specs=[
            pl.BlockSpec((N, din), lambda i: (0, 0)),
            pl.BlockSpec((din, dout), lambda i: (0, 0)),
            pl.BlockSpec((1, dout), lambda i: (0, 0)),
            pl.BlockSpec(memory_space=pl.ANY),
        ],
        out_specs=[
            pl.BlockSpec((N, dout), lambda i: (0, 0)),
            pl.BlockSpec((_CH, dout), lambda i: (i, 0)),
            pl.BlockSpec((_CH, dout), lambda i: (i, 0)),
        ],
        out_shape=[
            jax.ShapeDtypeStruct((N, dout), jnp.float32),
            jax.ShapeDtypeStruct((N, dout), jnp.float32),
            jax.ShapeDtypeStruct((N, dout), jnp.float32),
        ],
        scratch_shapes=[
            pltpu.VMEM((_NBUF, _CH, N), jnp.float32),
            pltpu.SemaphoreType.DMA((_NBUF,)),
        ],
    )(x, W, b2, adj)

    return (support, hidden, output)
